# baseline (speedup 1.0000x reference)
import functools

import numpy as np
import jax
import jax.numpy as jnp
from jax import lax
from jax.experimental import pallas as pl
from jax.experimental.pallas import tpu as pltpu
from jax.experimental.pallas import tpu_sc as plsc

_B = 4
_N = 16384
_C = 64
_M = 1024
_K = 32
_R2 = np.float32(0.2 * 0.2)
_D = 128
_NS = 128
_NL = 128
_RM = 128
_TQ = 512
_NT = _N // _TQ
_PT = 512
_P = _B * _M * _K
_NW = 32
_GCH = 128


_FB = 2


def _fps_body(pts_ref, dyn_ref, cen_ref, dist_ref):
    lin = (lax.broadcasted_iota(jnp.int32, (_NS, _NL), 0) * _NL
           + lax.broadcasted_iota(jnp.int32, (_NS, _NL), 1))
    lane8 = lax.broadcasted_iota(jnp.int32, (8, _NL), 1)
    sel0 = lin == 0
    carry0 = []
    for b in range(_FB):
        dist_ref[b] = jnp.full((_NS, _NL), 1e10, jnp.float32)
        for c in range(3):
            v = jnp.max(jnp.where(sel0, pts_ref[b, c], -jnp.inf))
            cen_ref[b, c, 0] = v
            carry0.append(v)

    def body(i, carry):
        new = []
        for b in range(_FB):
            cx, cy, cz = carry[3 * b:3 * b + 3]
            px = pts_ref[b, 0]
            py = pts_ref[b, 1]
            pz = pts_ref[b, 2]
            dx = px - cx
            dy = py - cy
            dz = pz - cz
            d = (dx * dx + dy * dy) + dz * dz
            dm = jnp.minimum(dist_ref[b], d)
            dist_ref[b] = dm
            mx = jnp.max(dm)
            nxt = jnp.min(jnp.where(dm == mx, lin, _N))
            s8 = nxt // _NL
            l = nxt - s8 * _NL
            row = dyn_ref[b, pl.ds(s8, 1)][0]
            sel = jnp.where(lane8 == l, row, -jnp.inf)
            ncx = jnp.max(sel[0:1])
            ncy = jnp.max(sel[1:2])
            ncz = jnp.max(sel[2:3])
            cen_ref[b, 0, i] = ncx
            cen_ref[b, 1, i] = ncy
            cen_ref[b, 2, i] = ncz
            new += [ncx, ncy, ncz]
        return tuple(new)

    lax.fori_loop(1, _M, body, tuple(carry0))


def _fps(pts4, pts_dyn):
    ngf = _B // _FB
    return pl.pallas_call(
        _fps_body,
        grid=(ngf,),
        out_shape=jax.ShapeDtypeStruct((_B, 3, _M), jnp.float32),
        in_specs=[
            pl.BlockSpec((_FB, 3, _NS, _NL), lambda g: (g, 0, 0, 0)),
            pl.BlockSpec((_FB, _NS, 8, _NL), lambda g: (g, 0, 0, 0)),
        ],
        out_specs=pl.BlockSpec((_FB, 3, _M), lambda g: (g, 0, 0),
                               memory_space=pltpu.SMEM),
        scratch_shapes=[pltpu.VMEM((_FB, _NS, _NL), jnp.float32)],
        compiler_params=pltpu.CompilerParams(
            dimension_semantics=("parallel",)),
    )(pts4, pts_dyn)


def _bq_body(cen_ref, pts_ref, out_ref):
    b = pl.program_id(0)
    c8 = cen_ref[...]
    cx = c8[:, 0:1]
    cy = c8[:, 1:2]
    cz = c8[:, 2:3]
    cn = (cx * cx + cy * cy) + cz * cz
    tri = (lax.broadcasted_iota(jnp.int32, (_TQ, _TQ), 0)
           <= lax.broadcasted_iota(jnp.int32, (_TQ, _TQ), 1)
           ).astype(jnp.float32)

    def tile_step(t, carry):
        def run(c):
            base, cnt = c
            p8 = pts_ref[0, t]
            s = lax.dot_general(c8, p8, (((1,), (0,)), ((), ())),
                                preferred_element_type=jnp.float32)
            px = p8[0:1, :]
            py = p8[1:2, :]
            pz = p8[2:3, :]
            pn = (px * px + py * py) + pz * pz
            d2 = (cn + pn) - 2.0 * s
            mi = (d2 <= _R2).astype(jnp.float32)
            gr = jnp.dot(mi, tri,
                         preferred_element_type=jnp.float32) + base
            adds = [jnp.sum((gr < np.float32(j + 1)).astype(jnp.float32),
                            axis=1, keepdims=True) for j in range(_K)]
            cnt = cnt + jnp.concatenate(adds, axis=1)
            base = gr[:, _TQ - 1:_TQ]
            return base, cnt

        base, cnt = carry
        return lax.cond(jnp.min(base) < np.float32(_K), run,
                        lambda c: c, (base, cnt))

    base0 = jnp.zeros((_RM, 1), jnp.float32)
    cnt0 = jnp.zeros((_RM, _K), jnp.float32)
    _, cnt = lax.fori_loop(0, _NT, tile_step, (base0, cnt0))
    cnti = cnt.astype(jnp.int32)
    c0 = cnti[:, 0:1]
    fallback = jnp.where(c0 < _N, c0, _N - 1)
    idx = jnp.where(cnti < _N, cnti, fallback)
    out_ref[...] = idx + b * _N


def _ball_query(cen8, pts_t):
    nmb = _M // _RM
    return pl.pallas_call(
        _bq_body,
        grid=(_B, nmb),
        in_specs=[
            pl.BlockSpec((_RM, 8), lambda b, m: (b * nmb + m, 0)),
            pl.BlockSpec((1, _NT, 8, _TQ), lambda b, m: (b, 0, 0, 0)),
        ],
        out_specs=pl.BlockSpec((_RM, _K), lambda b, m: (b * nmb + m, 0)),
        out_shape=jax.ShapeDtypeStruct((_B * _M, _K), jnp.int32),
        compiler_params=pltpu.CompilerParams(
            dimension_semantics=("parallel", "parallel")),
    )(cen8, pts_t)


def _sc_gather(table, gidx):
    n_rows = gidx.shape[0]
    per_w = n_rows // _NW
    n_chunks = per_w // _GCH
    mesh = plsc.VectorSubcoreMesh(core_axis_name="c", subcore_axis_name="s")

    @functools.partial(
        pl.kernel, mesh=mesh,
        out_type=jax.ShapeDtypeStruct((n_rows, _D), jnp.float32),
        scratch_types=[
            pltpu.VMEM((per_w,), jnp.int32),
            pltpu.VMEM((_GCH, _D), jnp.float32),
            pltpu.VMEM((_GCH, _D), jnp.float32),
            pltpu.SemaphoreType.DMA,
            pltpu.SemaphoreType.DMA,
        ],
    )
    def k(table_hbm, idx_hbm, out_hbm, idx_v, buf0, buf1, sem0, sem1):
        wid = lax.axis_index("s") * 2 + lax.axis_index("c")
        base = wid * per_w
        pltpu.sync_copy(idx_hbm.at[pl.ds(base, per_w)], idx_v)
        bufs = (buf0, buf1)
        sems = (sem0, sem1)
        if n_chunks == 1:
            pltpu.async_copy(table_hbm.at[idx_v.at[pl.ds(0, _GCH)]],
                             buf0, sem0).wait()
            pltpu.sync_copy(buf0, out_hbm.at[pl.ds(base, _GCH)])
        else:
            cp = pltpu.async_copy(table_hbm.at[idx_v.at[pl.ds(0, _GCH)]],
                                  buf0, sem0)
            for c in range(n_chunks):
                cur = bufs[c % 2]
                cp.wait()
                if c + 1 < n_chunks:
                    off = (c + 1) * _GCH
                    cp = pltpu.async_copy(
                        table_hbm.at[idx_v.at[pl.ds(off, _GCH)]],
                        bufs[(c + 1) % 2], sems[(c + 1) % 2])
                pltpu.sync_copy(cur, out_hbm.at[pl.ds(base + c * _GCH, _GCH)])

    return k(table, gidx)


def _mlp1_body(x_ref, cen_ref, w_ref, h_ref, st_ref):
    x = x_ref[...]
    w = w_ref[...]
    h = jnp.dot(x, w, preferred_element_type=jnp.float32)
    c8 = cen_ref[...]
    nm = _PT // _K
    corr = jnp.dot(c8, w[0:8, :], preferred_element_type=jnp.float32)
    corr = jnp.broadcast_to(corr[:, None, :], (nm, _K, 128)).reshape(_PT, 128)
    h = h - corr
    h_ref[...] = h

    @pl.when(pl.program_id(0) == 0)
    def _():
        st_ref[...] = jnp.zeros_like(st_ref)

    s = jnp.sum(h, axis=0, keepdims=True)
    s2 = jnp.sum(h * h, axis=0, keepdims=True)
    st_ref[...] += jnp.concatenate(
        [s, s2, jnp.zeros((6, 128), jnp.float32)], axis=0)


def _mlp_mid_body(h_ref, stin_ref, pr_ref, bias_ref, w_ref, o_ref, st_ref,
                  *, nout):
    stats = stin_ref[...]
    inv_p = np.float32(1.0 / _P)
    mean = stats[0:1, :] * inv_p
    var = stats[1:2, :] * inv_p - mean * mean
    g = pr_ref[0:1, :]
    be = pr_ref[1:2, :]
    bias = bias_ref[0:1, :]
    scale = g / jnp.sqrt(var + np.float32(1e-5))
    shift = be - mean * scale
    x = h_ref[...]
    xr = jnp.maximum(x * scale + shift, 0.0)
    o = jnp.dot(xr, w_ref[...], preferred_element_type=jnp.float32) + bias
    o_ref[...] = o

    @pl.when(pl.program_id(0) == 0)
    def _():
        st_ref[...] = jnp.zeros_like(st_ref)

    s = jnp.sum(o, axis=0, keepdims=True)
    s2 = jnp.sum(o * o, axis=0, keepdims=True)
    st_ref[...] += jnp.concatenate(
        [s, s2, jnp.zeros((6, nout), jnp.float32)], axis=0)


def _mlp3_body(h_ref, stin_ref, pr_ref, bias_ref, w_ref, gx_ref, gn_ref,
               st_ref):
    stats = stin_ref[...]
    inv_p = np.float32(1.0 / _P)
    mean = stats[0:1, :] * inv_p
    var = stats[1:2, :] * inv_p - mean * mean
    g = pr_ref[0:1, :]
    be = pr_ref[1:2, :]
    bias = bias_ref[0:1, :]
    scale = g / jnp.sqrt(var + np.float32(1e-5))
    shift = be - mean * scale
    x = h_ref[...]
    xr = jnp.maximum(x * scale + shift, 0.0)
    o = jnp.dot(xr, w_ref[...], preferred_element_type=jnp.float32) + bias
    og = o.reshape(_PT // _K, _K, 256)
    gx_ref[...] = jnp.max(og, axis=1)
    gn_ref[...] = jnp.min(og, axis=1)

    @pl.when(pl.program_id(0) == 0)
    def _():
        st_ref[...] = jnp.zeros_like(st_ref)

    s = jnp.sum(o, axis=0, keepdims=True)
    s2 = jnp.sum(o * o, axis=0, keepdims=True)
    st_ref[...] += jnp.concatenate(
        [s, s2, jnp.zeros((6, 256), jnp.float32)], axis=0)


def _mlp4_body(gx_ref, gn_ref, stin_ref, pr_ref, o_ref):
    stats = stin_ref[...]
    inv_p = np.float32(1.0 / _P)
    mean = stats[0:1, :] * inv_p
    var = stats[1:2, :] * inv_p - mean * mean
    g = pr_ref[0:1, :]
    be = pr_ref[1:2, :]
    scale = g / jnp.sqrt(var + np.float32(1e-5))
    shift = be - mean * scale
    sel = jnp.where(scale >= 0.0, gx_ref[...], gn_ref[...])
    o_ref[...] = jnp.maximum(sel * scale + shift, 0.0)


def _mlp(x0, cen8, w0t, w1t, w2t, pr1, pr2, pr3, b1r, b2r):
    ng = _P // _PT
    h1, st1 = pl.pallas_call(
        _mlp1_body,
        grid=(ng,),
        in_specs=[
            pl.BlockSpec((_PT, _D), lambda i: (i, 0)),
            pl.BlockSpec((_PT // _K, 8), lambda i: (i, 0)),
            pl.BlockSpec((_D, 128), lambda i: (0, 0)),
        ],
        out_specs=[
            pl.BlockSpec((_PT, 128), lambda i: (i, 0)),
            pl.BlockSpec((8, 128), lambda i: (0, 0)),
        ],
        out_shape=[
            jax.ShapeDtypeStruct((_P, 128), jnp.float32),
            jax.ShapeDtypeStruct((8, 128), jnp.float32),
        ],
    )(x0, cen8, w0t)

    def mid(h, st, pr, bias, wt, nout):
        return pl.pallas_call(
            functools.partial(_mlp_mid_body, nout=nout),
            grid=(ng,),
            in_specs=[
                pl.BlockSpec((_PT, 128), lambda i: (i, 0)),
                pl.BlockSpec((8, 128), lambda i: (0, 0)),
                pl.BlockSpec((8, 128), lambda i: (0, 0)),
                pl.BlockSpec((8, nout), lambda i: (0, 0)),
                pl.BlockSpec((128, nout), lambda i: (0, 0)),
            ],
            out_specs=[
                pl.BlockSpec((_PT, nout), lambda i: (i, 0)),
                pl.BlockSpec((8, nout), lambda i: (0, 0)),
            ],
            out_shape=[
                jax.ShapeDtypeStruct((_P, nout), jnp.float32),
                jax.ShapeDtypeStruct((8, nout), jnp.float32),
            ],
        )(h, st, pr, bias, wt)

    h2, st2 = mid(h1, st1, pr1, b1r, w1t, 128)

    gx, gn, st3 = pl.pallas_call(
        _mlp3_body,
        grid=(ng,),
        in_specs=[
            pl.BlockSpec((_PT, 128), lambda i: (i, 0)),
            pl.BlockSpec((8, 128), lambda i: (0, 0)),
            pl.BlockSpec((8, 128), lambda i: (0, 0)),
            pl.BlockSpec((8, 256), lambda i: (0, 0)),
            pl.BlockSpec((128, 256), lambda i: (0, 0)),
        ],
        out_specs=[
            pl.BlockSpec((_PT // _K, 256), lambda i: (i, 0)),
            pl.BlockSpec((_PT // _K, 256), lambda i: (i, 0)),
            pl.BlockSpec((8, 256), lambda i: (0, 0)),
        ],
        out_shape=[
            jax.ShapeDtypeStruct((_B * _M, 256), jnp.float32),
            jax.ShapeDtypeStruct((_B * _M, 256), jnp.float32),
            jax.ShapeDtypeStruct((8, 256), jnp.float32),
        ],
    )(h2, st2, pr2, b2r, w2t)

    ng4 = (_B * _M) // _PT
    pooled = pl.pallas_call(
        _mlp4_body,
        grid=(ng4,),
        in_specs=[
            pl.BlockSpec((_PT, 256), lambda i: (i, 0)),
            pl.BlockSpec((_PT, 256), lambda i: (i, 0)),
            pl.BlockSpec((8, 256), lambda i: (0, 0)),
            pl.BlockSpec((8, 256), lambda i: (0, 0)),
        ],
        out_specs=pl.BlockSpec((_PT, 256), lambda i: (i, 0)),
        out_shape=jax.ShapeDtypeStruct((_B * _M, 256), jnp.float32),
    )(gx, gn, st3, pr3)
    return pooled


def kernel(points, features, W0, b0, gamma0, beta0, W1, b1, gamma1, beta1,
           W2, b2, gamma2, beta2):
    f32 = jnp.float32
    pts_tr = jnp.swapaxes(points, 1, 2).reshape(_B * _N, 3)
    fts_tr = jnp.swapaxes(features, 1, 2).reshape(_B * _N, _C)
    table = jnp.concatenate(
        [pts_tr, fts_tr,
         jnp.ones((_B * _N, 1), f32),
         jnp.zeros((_B * _N, _D - _C - 4), f32)], axis=1)

    pts4 = points.reshape(_B, 3, _NS, _NL)
    pts_dyn = jnp.concatenate(
        [jnp.transpose(pts4, (0, 2, 1, 3)),
         jnp.zeros((_B, _NS, 5, _NL), f32)], axis=2)
    cen_xyz = _fps(pts4, pts_dyn)
    cen8 = jnp.concatenate(
        [jnp.transpose(cen_xyz, (0, 2, 1)).reshape(_B * _M, 3),
         jnp.zeros((_B * _M, 5), f32)], axis=1)

    pts_pad = jnp.concatenate(
        [points, jnp.zeros((_B, 5, _N), f32)], axis=1)
    pts_t = jnp.transpose(pts_pad.reshape(_B, 8, _NT, _TQ), (0, 2, 1, 3))
    idx = _ball_query(cen8, pts_t)

    x0 = _sc_gather(table, idx.reshape(_P))

    w0t = jnp.concatenate(
        [W0.T, b0[None, :], jnp.zeros((_D - 68, 128), f32)], axis=0)
    w1t = W1.T
    w2t = W2.T
    zr = jnp.zeros((6, 128), f32)
    zr2 = jnp.zeros((6, 256), f32)
    pr1 = jnp.concatenate([gamma0[None], beta0[None], zr], axis=0)
    pr2 = jnp.concatenate([gamma1[None], beta1[None], zr], axis=0)
    pr3 = jnp.concatenate([gamma2[None], beta2[None], zr2], axis=0)
    b1r = jnp.concatenate([b1[None], jnp.zeros((7, 128), f32)], axis=0)
    b2r = jnp.concatenate([b2[None], jnp.zeros((7, 256), f32)], axis=0)

    pooled = _mlp(x0, cen8, w0t, w1t, w2t, pr1, pr2, pr3, b1r, b2r)

    feat = jnp.transpose(pooled.reshape(_B, _M, 256), (0, 2, 1))
    return (cen_xyz, feat)


# speedup vs baseline: 1.0799x; 1.0799x over previous
import functools

import numpy as np
import jax
import jax.numpy as jnp
from jax import lax
from jax.experimental import pallas as pl
from jax.experimental.pallas import tpu as pltpu
from jax.experimental.pallas import tpu_sc as plsc

_B = 4
_N = 16384
_C = 64
_M = 1024
_K = 32
_R2 = np.float32(0.2 * 0.2)
_D = 128
_NS = 128
_NL = 128
_RM = 128
_TQ = 512
_NT = _N // _TQ
_PT = 512
_P = _B * _M * _K
_NW = 32
_GCH = 128


def _fps_body(pts_ref, dyn_ref, cen_ref):
    lin = (lax.broadcasted_iota(jnp.int32, (_NS, _NL), 0) * _NL
           + lax.broadcasted_iota(jnp.int32, (_NS, _NL), 1))
    lane8 = lax.broadcasted_iota(jnp.int32, (8, _NL), 1)
    sel0 = lin == 0
    coords0 = []
    dists0 = []
    for b in range(_B):
        dists0.append(jnp.full((_NS, _NL), 1e10, jnp.float32))
        for c in range(3):
            v = jnp.max(jnp.where(sel0, pts_ref[b, c], -jnp.inf))
            cen_ref[b, c, 0] = v
            coords0.append(v)

    def step(i, dists, coords):
        nd = []
        nc = []
        for b in range(_B):
            cx, cy, cz = coords[3 * b:3 * b + 3]
            px = pts_ref[b, 0]
            py = pts_ref[b, 1]
            pz = pts_ref[b, 2]
            dx = px - cx
            dy = py - cy
            dz = pz - cz
            d = (dx * dx + dy * dy) + dz * dz
            dm = jnp.minimum(dists[b], d)
            mx = jnp.max(dm)
            nxt = jnp.min(jnp.where(dm == mx, lin, _N))
            s8 = nxt // _NL
            l = nxt - s8 * _NL
            row = dyn_ref[b, pl.ds(s8, 1)][0]
            sel = jnp.where(lane8 == l, row, -jnp.inf)
            ncx = jnp.max(sel[0:1])
            ncy = jnp.max(sel[1:2])
            ncz = jnp.max(sel[2:3])
            cen_ref[b, 0, i] = ncx
            cen_ref[b, 1, i] = ncy
            cen_ref[b, 2, i] = ncz
            nd.append(dm)
            nc += [ncx, ncy, ncz]
        return nd, nc

    def body2(t, carry):
        dists = list(carry[:_B])
        coords = list(carry[_B:])
        i = 2 * t + 1
        dists, coords = step(i, dists, coords)
        dists, coords = step(i + 1, dists, coords)
        return tuple(dists) + tuple(coords)

    carry = lax.fori_loop(0, (_M - 2) // 2, body2,
                          tuple(dists0) + tuple(coords0))
    step(_M - 1, list(carry[:_B]), list(carry[_B:]))


def _fps(pts4, pts_dyn):
    return pl.pallas_call(
        _fps_body,
        out_shape=jax.ShapeDtypeStruct((_B, 3, _M), jnp.float32),
        in_specs=[pl.BlockSpec(memory_space=pltpu.VMEM),
                  pl.BlockSpec(memory_space=pltpu.VMEM)],
        out_specs=pl.BlockSpec(memory_space=pltpu.SMEM),
    )(pts4, pts_dyn)


def _bq_body(cen_ref, pts_ref, out_ref):
    b = pl.program_id(0)
    c8 = cen_ref[...]
    cx = c8[:, 0:1]
    cy = c8[:, 1:2]
    cz = c8[:, 2:3]
    cn = (cx * cx + cy * cy) + cz * cz
    tri = (lax.broadcasted_iota(jnp.int32, (_TQ, _TQ), 0)
           <= lax.broadcasted_iota(jnp.int32, (_TQ, _TQ), 1)
           ).astype(jnp.float32)

    def tile_step(t, carry):
        def run(c):
            base, cnt = c
            p8 = pts_ref[0, t]
            s = lax.dot_general(c8, p8, (((1,), (0,)), ((), ())),
                                preferred_element_type=jnp.float32)
            px = p8[0:1, :]
            py = p8[1:2, :]
            pz = p8[2:3, :]
            pn = (px * px + py * py) + pz * pz
            d2 = (cn + pn) - 2.0 * s
            mi = (d2 <= _R2).astype(jnp.float32)
            gr = jnp.dot(mi, tri,
                         preferred_element_type=jnp.float32) + base
            adds = [jnp.sum((gr < np.float32(j + 1)).astype(jnp.float32),
                            axis=1, keepdims=True) for j in range(_K)]
            cnt = cnt + jnp.concatenate(adds, axis=1)
            base = gr[:, _TQ - 1:_TQ]
            return base, cnt

        base, cnt = carry
        return lax.cond(jnp.min(base) < np.float32(_K), run,
                        lambda c: c, (base, cnt))

    base0 = jnp.zeros((_RM, 1), jnp.float32)
    cnt0 = jnp.zeros((_RM, _K), jnp.float32)
    _, cnt = lax.fori_loop(0, _NT, tile_step, (base0, cnt0))
    cnti = cnt.astype(jnp.int32)
    c0 = cnti[:, 0:1]
    fallback = jnp.where(c0 < _N, c0, _N - 1)
    idx = jnp.where(cnti < _N, cnti, fallback)
    out_ref[...] = idx + b * _N


def _ball_query(cen8, pts_t):
    nmb = _M // _RM
    return pl.pallas_call(
        _bq_body,
        grid=(_B, nmb),
        in_specs=[
            pl.BlockSpec((_RM, 8), lambda b, m: (b * nmb + m, 0)),
            pl.BlockSpec((1, _NT, 8, _TQ), lambda b, m: (b, 0, 0, 0)),
        ],
        out_specs=pl.BlockSpec((_RM, _K), lambda b, m: (b * nmb + m, 0)),
        out_shape=jax.ShapeDtypeStruct((_B * _M, _K), jnp.int32),
    )(cen8, pts_t)


def _sc_gather(table, gidx):
    n_rows = gidx.shape[0]
    per_w = n_rows // _NW
    n_chunks = per_w // _GCH
    mesh = plsc.VectorSubcoreMesh(core_axis_name="c", subcore_axis_name="s")

    @functools.partial(
        pl.kernel, mesh=mesh,
        out_type=jax.ShapeDtypeStruct((n_rows, _D), jnp.float32),
        scratch_types=[
            pltpu.VMEM((per_w,), jnp.int32),
            pltpu.VMEM((_GCH, _D), jnp.float32),
            pltpu.VMEM((_GCH, _D), jnp.float32),
            pltpu.SemaphoreType.DMA,
            pltpu.SemaphoreType.DMA,
        ],
    )
    def k(table_hbm, idx_hbm, out_hbm, idx_v, buf0, buf1, sem0, sem1):
        wid = lax.axis_index("s") * 2 + lax.axis_index("c")
        base = wid * per_w
        pltpu.sync_copy(idx_hbm.at[pl.ds(base, per_w)], idx_v)
        bufs = (buf0, buf1)
        sems = (sem0, sem1)
        if n_chunks == 1:
            pltpu.async_copy(table_hbm.at[idx_v.at[pl.ds(0, _GCH)]],
                             buf0, sem0).wait()
            pltpu.sync_copy(buf0, out_hbm.at[pl.ds(base, _GCH)])
        else:
            cp = pltpu.async_copy(table_hbm.at[idx_v.at[pl.ds(0, _GCH)]],
                                  buf0, sem0)
            for c in range(n_chunks):
                cur = bufs[c % 2]
                cp.wait()
                if c + 1 < n_chunks:
                    off = (c + 1) * _GCH
                    cp = pltpu.async_copy(
                        table_hbm.at[idx_v.at[pl.ds(off, _GCH)]],
                        bufs[(c + 1) % 2], sems[(c + 1) % 2])
                pltpu.sync_copy(cur, out_hbm.at[pl.ds(base + c * _GCH, _GCH)])

    return k(table, gidx)


def _mlp1_body(x_ref, cen_ref, w_ref, h_ref, st_ref):
    x = x_ref[...]
    w = w_ref[...]
    h = jnp.dot(x, w, preferred_element_type=jnp.float32)
    c8 = cen_ref[...]
    nm = _PT // _K
    corr = jnp.dot(c8, w[0:8, :], preferred_element_type=jnp.float32)
    corr = jnp.broadcast_to(corr[:, None, :], (nm, _K, 128)).reshape(_PT, 128)
    h = h - corr
    h_ref[...] = h

    @pl.when(pl.program_id(0) == 0)
    def _():
        st_ref[...] = jnp.zeros_like(st_ref)

    s = jnp.sum(h, axis=0, keepdims=True)
    s2 = jnp.sum(h * h, axis=0, keepdims=True)
    st_ref[...] += jnp.concatenate(
        [s, s2, jnp.zeros((6, 128), jnp.float32)], axis=0)


def _mlp_mid_body(h_ref, stin_ref, pr_ref, bias_ref, w_ref, o_ref, st_ref,
                  *, nout):
    stats = stin_ref[...]
    inv_p = np.float32(1.0 / _P)
    mean = stats[0:1, :] * inv_p
    var = stats[1:2, :] * inv_p - mean * mean
    g = pr_ref[0:1, :]
    be = pr_ref[1:2, :]
    bias = bias_ref[0:1, :]
    scale = g / jnp.sqrt(var + np.float32(1e-5))
    shift = be - mean * scale
    x = h_ref[...]
    xr = jnp.maximum(x * scale + shift, 0.0)
    o = jnp.dot(xr, w_ref[...], preferred_element_type=jnp.float32) + bias
    o_ref[...] = o

    @pl.when(pl.program_id(0) == 0)
    def _():
        st_ref[...] = jnp.zeros_like(st_ref)

    s = jnp.sum(o, axis=0, keepdims=True)
    s2 = jnp.sum(o * o, axis=0, keepdims=True)
    st_ref[...] += jnp.concatenate(
        [s, s2, jnp.zeros((6, nout), jnp.float32)], axis=0)


def _mlp3_body(h_ref, stin_ref, pr_ref, bias_ref, w_ref, gx_ref, gn_ref,
               st_ref):
    stats = stin_ref[...]
    inv_p = np.float32(1.0 / _P)
    mean = stats[0:1, :] * inv_p
    var = stats[1:2, :] * inv_p - mean * mean
    g = pr_ref[0:1, :]
    be = pr_ref[1:2, :]
    bias = bias_ref[0:1, :]
    scale = g / jnp.sqrt(var + np.float32(1e-5))
    shift = be - mean * scale
    x = h_ref[...]
    xr = jnp.maximum(x * scale + shift, 0.0)
    o = jnp.dot(xr, w_ref[...], preferred_element_type=jnp.float32) + bias
    og = o.reshape(_PT // _K, _K, 256)
    gx_ref[...] = jnp.max(og, axis=1)
    gn_ref[...] = jnp.min(og, axis=1)

    @pl.when(pl.program_id(0) == 0)
    def _():
        st_ref[...] = jnp.zeros_like(st_ref)

    s = jnp.sum(o, axis=0, keepdims=True)
    s2 = jnp.sum(o * o, axis=0, keepdims=True)
    st_ref[...] += jnp.concatenate(
        [s, s2, jnp.zeros((6, 256), jnp.float32)], axis=0)


def _mlp4_body(gx_ref, gn_ref, stin_ref, pr_ref, o_ref):
    stats = stin_ref[...]
    inv_p = np.float32(1.0 / _P)
    mean = stats[0:1, :] * inv_p
    var = stats[1:2, :] * inv_p - mean * mean
    g = pr_ref[0:1, :]
    be = pr_ref[1:2, :]
    scale = g / jnp.sqrt(var + np.float32(1e-5))
    shift = be - mean * scale
    sel = jnp.where(scale >= 0.0, gx_ref[...], gn_ref[...])
    o_ref[...] = jnp.maximum(sel * scale + shift, 0.0)


def _mlp(x0, cen8, w0t, w1t, w2t, pr1, pr2, pr3, b1r, b2r):
    ng = _P // _PT
    h1, st1 = pl.pallas_call(
        _mlp1_body,
        grid=(ng,),
        in_specs=[
            pl.BlockSpec((_PT, _D), lambda i: (i, 0)),
            pl.BlockSpec((_PT // _K, 8), lambda i: (i, 0)),
            pl.BlockSpec((_D, 128), lambda i: (0, 0)),
        ],
        out_specs=[
            pl.BlockSpec((_PT, 128), lambda i: (i, 0)),
            pl.BlockSpec((8, 128), lambda i: (0, 0)),
        ],
        out_shape=[
            jax.ShapeDtypeStruct((_P, 128), jnp.float32),
            jax.ShapeDtypeStruct((8, 128), jnp.float32),
        ],
    )(x0, cen8, w0t)

    def mid(h, st, pr, bias, wt, nout):
        return pl.pallas_call(
            functools.partial(_mlp_mid_body, nout=nout),
            grid=(ng,),
            in_specs=[
                pl.BlockSpec((_PT, 128), lambda i: (i, 0)),
                pl.BlockSpec((8, 128), lambda i: (0, 0)),
                pl.BlockSpec((8, 128), lambda i: (0, 0)),
                pl.BlockSpec((8, nout), lambda i: (0, 0)),
                pl.BlockSpec((128, nout), lambda i: (0, 0)),
            ],
            out_specs=[
                pl.BlockSpec((_PT, nout), lambda i: (i, 0)),
                pl.BlockSpec((8, nout), lambda i: (0, 0)),
            ],
            out_shape=[
                jax.ShapeDtypeStruct((_P, nout), jnp.float32),
                jax.ShapeDtypeStruct((8, nout), jnp.float32),
            ],
        )(h, st, pr, bias, wt)

    h2, st2 = mid(h1, st1, pr1, b1r, w1t, 128)

    gx, gn, st3 = pl.pallas_call(
        _mlp3_body,
        grid=(ng,),
        in_specs=[
            pl.BlockSpec((_PT, 128), lambda i: (i, 0)),
            pl.BlockSpec((8, 128), lambda i: (0, 0)),
            pl.BlockSpec((8, 128), lambda i: (0, 0)),
            pl.BlockSpec((8, 256), lambda i: (0, 0)),
            pl.BlockSpec((128, 256), lambda i: (0, 0)),
        ],
        out_specs=[
            pl.BlockSpec((_PT // _K, 256), lambda i: (i, 0)),
            pl.BlockSpec((_PT // _K, 256), lambda i: (i, 0)),
            pl.BlockSpec((8, 256), lambda i: (0, 0)),
        ],
        out_shape=[
            jax.ShapeDtypeStruct((_B * _M, 256), jnp.float32),
            jax.ShapeDtypeStruct((_B * _M, 256), jnp.float32),
            jax.ShapeDtypeStruct((8, 256), jnp.float32),
        ],
    )(h2, st2, pr2, b2r, w2t)

    ng4 = (_B * _M) // _PT
    pooled = pl.pallas_call(
        _mlp4_body,
        grid=(ng4,),
        in_specs=[
            pl.BlockSpec((_PT, 256), lambda i: (i, 0)),
            pl.BlockSpec((_PT, 256), lambda i: (i, 0)),
            pl.BlockSpec((8, 256), lambda i: (0, 0)),
            pl.BlockSpec((8, 256), lambda i: (0, 0)),
        ],
        out_specs=pl.BlockSpec((_PT, 256), lambda i: (i, 0)),
        out_shape=jax.ShapeDtypeStruct((_B * _M, 256), jnp.float32),
    )(gx, gn, st3, pr3)
    return pooled


def kernel(points, features, W0, b0, gamma0, beta0, W1, b1, gamma1, beta1,
           W2, b2, gamma2, beta2):
    f32 = jnp.float32
    pts_tr = jnp.swapaxes(points, 1, 2).reshape(_B * _N, 3)
    fts_tr = jnp.swapaxes(features, 1, 2).reshape(_B * _N, _C)
    table = jnp.concatenate(
        [pts_tr, fts_tr,
         jnp.ones((_B * _N, 1), f32),
         jnp.zeros((_B * _N, _D - _C - 4), f32)], axis=1)

    pts4 = points.reshape(_B, 3, _NS, _NL)
    pts_dyn = jnp.concatenate(
        [jnp.transpose(pts4, (0, 2, 1, 3)),
         jnp.zeros((_B, _NS, 5, _NL), f32)], axis=2)
    cen_xyz = _fps(pts4, pts_dyn)
    cen8 = jnp.concatenate(
        [jnp.transpose(cen_xyz, (0, 2, 1)).reshape(_B * _M, 3),
         jnp.zeros((_B * _M, 5), f32)], axis=1)

    pts_pad = jnp.concatenate(
        [points, jnp.zeros((_B, 5, _N), f32)], axis=1)
    pts_t = jnp.transpose(pts_pad.reshape(_B, 8, _NT, _TQ), (0, 2, 1, 3))
    idx = _ball_query(cen8, pts_t)

    x0 = _sc_gather(table, idx.reshape(_P))

    w0t = jnp.concatenate(
        [W0.T, b0[None, :], jnp.zeros((_D - 68, 128), f32)], axis=0)
    w1t = W1.T
    w2t = W2.T
    zr = jnp.zeros((6, 128), f32)
    zr2 = jnp.zeros((6, 256), f32)
    pr1 = jnp.concatenate([gamma0[None], beta0[None], zr], axis=0)
    pr2 = jnp.concatenate([gamma1[None], beta1[None], zr], axis=0)
    pr3 = jnp.concatenate([gamma2[None], beta2[None], zr2], axis=0)
    b1r = jnp.concatenate([b1[None], jnp.zeros((7, 128), f32)], axis=0)
    b2r = jnp.concatenate([b2[None], jnp.zeros((7, 256), f32)], axis=0)

    pooled = _mlp(x0, cen8, w0t, w1t, w2t, pr1, pr2, pr3, b1r, b2r)

    feat = jnp.transpose(pooled.reshape(_B, _M, 256), (0, 2, 1))
    return (cen_xyz, feat)


# speedup vs baseline: 1.0804x; 1.0004x over previous
import functools

import numpy as np
import jax
import jax.numpy as jnp
from jax import lax
from jax.experimental import pallas as pl
from jax.experimental.pallas import tpu as pltpu
from jax.experimental.pallas import tpu_sc as plsc

_B = 4
_N = 16384
_C = 64
_M = 1024
_K = 32
_R2 = np.float32(0.2 * 0.2)
_D = 128
_NS = 128
_NL = 128
_RM = 128
_TQ = 512
_NT = _N // _TQ
_PT = 512
_P = _B * _M * _K
_NW = 32
_GCH = 128


def _fps_body(pts_ref, dyn_ref, *refs):
    cen_refs = refs[:_B]
    dist_refs = refs[_B:]
    lin = (lax.broadcasted_iota(jnp.int32, (_NS, _NL), 0) * _NL
           + lax.broadcasted_iota(jnp.int32, (_NS, _NL), 1))
    lane8 = lax.broadcasted_iota(jnp.int32, (8, _NL), 1)
    sel0 = lin == 0
    coords0 = []
    for b in range(_B):
        dist_refs[b][...] = jnp.full((_NS, _NL), 1e10, jnp.float32)
        for c in range(3):
            v = jnp.max(jnp.where(sel0, pts_ref[b, c], -jnp.inf))
            cen_refs[b][c, 0] = v
            coords0.append(v)

    def step(i, coords):
        nc = []
        for b in range(_B):
            cx, cy, cz = coords[3 * b:3 * b + 3]
            px = pts_ref[b, 0]
            py = pts_ref[b, 1]
            pz = pts_ref[b, 2]
            dx = px - cx
            dy = py - cy
            dz = pz - cz
            d = (dx * dx + dy * dy) + dz * dz
            dm = jnp.minimum(dist_refs[b][...], d)
            dist_refs[b][...] = dm
            mx = jnp.max(dm)
            nxt = jnp.min(jnp.where(dm == mx, lin, _N))
            s8 = nxt // _NL
            l = nxt - s8 * _NL
            row = dyn_ref[b, pl.ds(s8, 1)][0]
            sel = jnp.where(lane8 == l, row, -jnp.inf)
            ncx = jnp.max(sel[0:1])
            ncy = jnp.max(sel[1:2])
            ncz = jnp.max(sel[2:3])
            cen_refs[b][0, i] = ncx
            cen_refs[b][1, i] = ncy
            cen_refs[b][2, i] = ncz
            nc += [ncx, ncy, ncz]
        return nc

    def body2(t, coords):
        i = 2 * t + 1
        coords = step(i, list(coords))
        coords = step(i + 1, coords)
        return tuple(coords)

    coords = lax.fori_loop(0, (_M - 2) // 2, body2, tuple(coords0))
    step(_M - 1, list(coords))


def _fps(pts4, pts_dyn):
    cens = pl.pallas_call(
        _fps_body,
        out_shape=[jax.ShapeDtypeStruct((3, _M), jnp.float32)] * _B,
        in_specs=[pl.BlockSpec(memory_space=pltpu.VMEM),
                  pl.BlockSpec(memory_space=pltpu.VMEM)],
        out_specs=[pl.BlockSpec(memory_space=pltpu.SMEM)] * _B,
        scratch_shapes=[pltpu.VMEM((_NS, _NL), jnp.float32)] * _B,
    )(pts4, pts_dyn)
    return jnp.stack(cens, axis=0)


def _bq_body(cen_ref, pts_ref, out_ref):
    b = pl.program_id(0)
    c8 = cen_ref[...]
    cx = c8[:, 0:1]
    cy = c8[:, 1:2]
    cz = c8[:, 2:3]
    cn = (cx * cx + cy * cy) + cz * cz
    tri = (lax.broadcasted_iota(jnp.int32, (_TQ, _TQ), 0)
           <= lax.broadcasted_iota(jnp.int32, (_TQ, _TQ), 1)
           ).astype(jnp.float32)

    def tile_step(t, carry):
        def run(c):
            base, cnt = c
            p8 = pts_ref[0, t]
            s = lax.dot_general(c8, p8, (((1,), (0,)), ((), ())),
                                preferred_element_type=jnp.float32)
            px = p8[0:1, :]
            py = p8[1:2, :]
            pz = p8[2:3, :]
            pn = (px * px + py * py) + pz * pz
            d2 = (cn + pn) - 2.0 * s
            mi = (d2 <= _R2).astype(jnp.float32)
            gr = jnp.dot(mi, tri,
                         preferred_element_type=jnp.float32) + base
            adds = [jnp.sum((gr < np.float32(j + 1)).astype(jnp.float32),
                            axis=1, keepdims=True) for j in range(_K)]
            cnt = cnt + jnp.concatenate(adds, axis=1)
            base = gr[:, _TQ - 1:_TQ]
            return base, cnt

        base, cnt = carry
        return lax.cond(jnp.min(base) < np.float32(_K), run,
                        lambda c: c, (base, cnt))

    base0 = jnp.zeros((_RM, 1), jnp.float32)
    cnt0 = jnp.zeros((_RM, _K), jnp.float32)
    _, cnt = lax.fori_loop(0, _NT, tile_step, (base0, cnt0))
    cnti = cnt.astype(jnp.int32)
    c0 = cnti[:, 0:1]
    fallback = jnp.where(c0 < _N, c0, _N - 1)
    idx = jnp.where(cnti < _N, cnti, fallback)
    out_ref[...] = idx + b * _N


def _ball_query(cen8, pts_t):
    nmb = _M // _RM
    return pl.pallas_call(
        _bq_body,
        grid=(_B, nmb),
        in_specs=[
            pl.BlockSpec((_RM, 8), lambda b, m: (b * nmb + m, 0)),
            pl.BlockSpec((1, _NT, 8, _TQ), lambda b, m: (b, 0, 0, 0)),
        ],
        out_specs=pl.BlockSpec((_RM, _K), lambda b, m: (b * nmb + m, 0)),
        out_shape=jax.ShapeDtypeStruct((_B * _M, _K), jnp.int32),
    )(cen8, pts_t)


def _sc_gather(table, gidx):
    n_rows = gidx.shape[0]
    per_w = n_rows // _NW
    n_chunks = per_w // _GCH
    mesh = plsc.VectorSubcoreMesh(core_axis_name="c", subcore_axis_name="s")

    @functools.partial(
        pl.kernel, mesh=mesh,
        out_type=jax.ShapeDtypeStruct((n_rows, _D), jnp.float32),
        scratch_types=[
            pltpu.VMEM((per_w,), jnp.int32),
            pltpu.VMEM((_GCH, _D), jnp.float32),
            pltpu.VMEM((_GCH, _D), jnp.float32),
            pltpu.SemaphoreType.DMA,
            pltpu.SemaphoreType.DMA,
        ],
    )
    def k(table_hbm, idx_hbm, out_hbm, idx_v, buf0, buf1, sem0, sem1):
        wid = lax.axis_index("s") * 2 + lax.axis_index("c")
        base = wid * per_w
        pltpu.sync_copy(idx_hbm.at[pl.ds(base, per_w)], idx_v)
        bufs = (buf0, buf1)
        sems = (sem0, sem1)
        if n_chunks == 1:
            pltpu.async_copy(table_hbm.at[idx_v.at[pl.ds(0, _GCH)]],
                             buf0, sem0).wait()
            pltpu.sync_copy(buf0, out_hbm.at[pl.ds(base, _GCH)])
        else:
            cp = pltpu.async_copy(table_hbm.at[idx_v.at[pl.ds(0, _GCH)]],
                                  buf0, sem0)
            for c in range(n_chunks):
                cur = bufs[c % 2]
                cp.wait()
                if c + 1 < n_chunks:
                    off = (c + 1) * _GCH
                    cp = pltpu.async_copy(
                        table_hbm.at[idx_v.at[pl.ds(off, _GCH)]],
                        bufs[(c + 1) % 2], sems[(c + 1) % 2])
                pltpu.sync_copy(cur, out_hbm.at[pl.ds(base + c * _GCH, _GCH)])

    return k(table, gidx)


def _mlp1_body(x_ref, cen_ref, w_ref, h_ref, st_ref):
    x = x_ref[...]
    w = w_ref[...]
    h = jnp.dot(x, w, preferred_element_type=jnp.float32)
    c8 = cen_ref[...]
    nm = _PT // _K
    corr = jnp.dot(c8, w[0:8, :], preferred_element_type=jnp.float32)
    corr = jnp.broadcast_to(corr[:, None, :], (nm, _K, 128)).reshape(_PT, 128)
    h = h - corr
    h_ref[...] = h

    @pl.when(pl.program_id(0) == 0)
    def _():
        st_ref[...] = jnp.zeros_like(st_ref)

    s = jnp.sum(h, axis=0, keepdims=True)
    s2 = jnp.sum(h * h, axis=0, keepdims=True)
    st_ref[...] += jnp.concatenate(
        [s, s2, jnp.zeros((6, 128), jnp.float32)], axis=0)


def _mlp_mid_body(h_ref, stin_ref, pr_ref, bias_ref, w_ref, o_ref, st_ref,
                  *, nout):
    stats = stin_ref[...]
    inv_p = np.float32(1.0 / _P)
    mean = stats[0:1, :] * inv_p
    var = stats[1:2, :] * inv_p - mean * mean
    g = pr_ref[0:1, :]
    be = pr_ref[1:2, :]
    bias = bias_ref[0:1, :]
    scale = g / jnp.sqrt(var + np.float32(1e-5))
    shift = be - mean * scale
    x = h_ref[...]
    xr = jnp.maximum(x * scale + shift, 0.0)
    o = jnp.dot(xr, w_ref[...], preferred_element_type=jnp.float32) + bias
    o_ref[...] = o

    @pl.when(pl.program_id(0) == 0)
    def _():
        st_ref[...] = jnp.zeros_like(st_ref)

    s = jnp.sum(o, axis=0, keepdims=True)
    s2 = jnp.sum(o * o, axis=0, keepdims=True)
    st_ref[...] += jnp.concatenate(
        [s, s2, jnp.zeros((6, nout), jnp.float32)], axis=0)


def _mlp3_body(h_ref, stin_ref, pr_ref, bias_ref, w_ref, gx_ref, gn_ref,
               st_ref):
    stats = stin_ref[...]
    inv_p = np.float32(1.0 / _P)
    mean = stats[0:1, :] * inv_p
    var = stats[1:2, :] * inv_p - mean * mean
    g = pr_ref[0:1, :]
    be = pr_ref[1:2, :]
    bias = bias_ref[0:1, :]
    scale = g / jnp.sqrt(var + np.float32(1e-5))
    shift = be - mean * scale
    x = h_ref[...]
    xr = jnp.maximum(x * scale + shift, 0.0)
    o = jnp.dot(xr, w_ref[...], preferred_element_type=jnp.float32) + bias
    og = o.reshape(_PT // _K, _K, 256)
    gx_ref[...] = jnp.max(og, axis=1)
    gn_ref[...] = jnp.min(og, axis=1)

    @pl.when(pl.program_id(0) == 0)
    def _():
        st_ref[...] = jnp.zeros_like(st_ref)

    s = jnp.sum(o, axis=0, keepdims=True)
    s2 = jnp.sum(o * o, axis=0, keepdims=True)
    st_ref[...] += jnp.concatenate(
        [s, s2, jnp.zeros((6, 256), jnp.float32)], axis=0)


def _mlp4_body(gx_ref, gn_ref, stin_ref, pr_ref, o_ref):
    stats = stin_ref[...]
    inv_p = np.float32(1.0 / _P)
    mean = stats[0:1, :] * inv_p
    var = stats[1:2, :] * inv_p - mean * mean
    g = pr_ref[0:1, :]
    be = pr_ref[1:2, :]
    scale = g / jnp.sqrt(var + np.float32(1e-5))
    shift = be - mean * scale
    sel = jnp.where(scale >= 0.0, gx_ref[...], gn_ref[...])
    o_ref[...] = jnp.maximum(sel * scale + shift, 0.0)


def _mlp(x0, cen8, w0t, w1t, w2t, pr1, pr2, pr3, b1r, b2r):
    ng = _P // _PT
    h1, st1 = pl.pallas_call(
        _mlp1_body,
        grid=(ng,),
        in_specs=[
            pl.BlockSpec((_PT, _D), lambda i: (i, 0)),
            pl.BlockSpec((_PT // _K, 8), lambda i: (i, 0)),
            pl.BlockSpec((_D, 128), lambda i: (0, 0)),
        ],
        out_specs=[
            pl.BlockSpec((_PT, 128), lambda i: (i, 0)),
            pl.BlockSpec((8, 128), lambda i: (0, 0)),
        ],
        out_shape=[
            jax.ShapeDtypeStruct((_P, 128), jnp.float32),
            jax.ShapeDtypeStruct((8, 128), jnp.float32),
        ],
    )(x0, cen8, w0t)

    def mid(h, st, pr, bias, wt, nout):
        return pl.pallas_call(
            functools.partial(_mlp_mid_body, nout=nout),
            grid=(ng,),
            in_specs=[
                pl.BlockSpec((_PT, 128), lambda i: (i, 0)),
                pl.BlockSpec((8, 128), lambda i: (0, 0)),
                pl.BlockSpec((8, 128), lambda i: (0, 0)),
                pl.BlockSpec((8, nout), lambda i: (0, 0)),
                pl.BlockSpec((128, nout), lambda i: (0, 0)),
            ],
            out_specs=[
                pl.BlockSpec((_PT, nout), lambda i: (i, 0)),
                pl.BlockSpec((8, nout), lambda i: (0, 0)),
            ],
            out_shape=[
                jax.ShapeDtypeStruct((_P, nout), jnp.float32),
                jax.ShapeDtypeStruct((8, nout), jnp.float32),
            ],
        )(h, st, pr, bias, wt)

    h2, st2 = mid(h1, st1, pr1, b1r, w1t, 128)

    gx, gn, st3 = pl.pallas_call(
        _mlp3_body,
        grid=(ng,),
        in_specs=[
            pl.BlockSpec((_PT, 128), lambda i: (i, 0)),
            pl.BlockSpec((8, 128), lambda i: (0, 0)),
            pl.BlockSpec((8, 128), lambda i: (0, 0)),
            pl.BlockSpec((8, 256), lambda i: (0, 0)),
            pl.BlockSpec((128, 256), lambda i: (0, 0)),
        ],
        out_specs=[
            pl.BlockSpec((_PT // _K, 256), lambda i: (i, 0)),
            pl.BlockSpec((_PT // _K, 256), lambda i: (i, 0)),
            pl.BlockSpec((8, 256), lambda i: (0, 0)),
        ],
        out_shape=[
            jax.ShapeDtypeStruct((_B * _M, 256), jnp.float32),
            jax.ShapeDtypeStruct((_B * _M, 256), jnp.float32),
            jax.ShapeDtypeStruct((8, 256), jnp.float32),
        ],
    )(h2, st2, pr2, b2r, w2t)

    ng4 = (_B * _M) // _PT
    pooled = pl.pallas_call(
        _mlp4_body,
        grid=(ng4,),
        in_specs=[
            pl.BlockSpec((_PT, 256), lambda i: (i, 0)),
            pl.BlockSpec((_PT, 256), lambda i: (i, 0)),
            pl.BlockSpec((8, 256), lambda i: (0, 0)),
            pl.BlockSpec((8, 256), lambda i: (0, 0)),
        ],
        out_specs=pl.BlockSpec((_PT, 256), lambda i: (i, 0)),
        out_shape=jax.ShapeDtypeStruct((_B * _M, 256), jnp.float32),
    )(gx, gn, st3, pr3)
    return pooled


def kernel(points, features, W0, b0, gamma0, beta0, W1, b1, gamma1, beta1,
           W2, b2, gamma2, beta2):
    f32 = jnp.float32
    pts_tr = jnp.swapaxes(points, 1, 2).reshape(_B * _N, 3)
    fts_tr = jnp.swapaxes(features, 1, 2).reshape(_B * _N, _C)
    table = jnp.concatenate(
        [pts_tr, fts_tr,
         jnp.ones((_B * _N, 1), f32),
         jnp.zeros((_B * _N, _D - _C - 4), f32)], axis=1)

    pts4 = points.reshape(_B, 3, _NS, _NL)
    pts_dyn = jnp.concatenate(
        [jnp.transpose(pts4, (0, 2, 1, 3)),
         jnp.zeros((_B, _NS, 5, _NL), f32)], axis=2)
    cen_xyz = _fps(pts4, pts_dyn)
    cen8 = jnp.concatenate(
        [jnp.transpose(cen_xyz, (0, 2, 1)).reshape(_B * _M, 3),
         jnp.zeros((_B * _M, 5), f32)], axis=1)

    pts_pad = jnp.concatenate(
        [points, jnp.zeros((_B, 5, _N), f32)], axis=1)
    pts_t = jnp.transpose(pts_pad.reshape(_B, 8, _NT, _TQ), (0, 2, 1, 3))
    idx = _ball_query(cen8, pts_t)

    x0 = _sc_gather(table, idx.reshape(_P))

    w0t = jnp.concatenate(
        [W0.T, b0[None, :], jnp.zeros((_D - 68, 128), f32)], axis=0)
    w1t = W1.T
    w2t = W2.T
    zr = jnp.zeros((6, 128), f32)
    zr2 = jnp.zeros((6, 256), f32)
    pr1 = jnp.concatenate([gamma0[None], beta0[None], zr], axis=0)
    pr2 = jnp.concatenate([gamma1[None], beta1[None], zr], axis=0)
    pr3 = jnp.concatenate([gamma2[None], beta2[None], zr2], axis=0)
    b1r = jnp.concatenate([b1[None], jnp.zeros((7, 128), f32)], axis=0)
    b2r = jnp.concatenate([b2[None], jnp.zeros((7, 256), f32)], axis=0)

    pooled = _mlp(x0, cen8, w0t, w1t, w2t, pr1, pr2, pr3, b1r, b2r)

    feat = jnp.transpose(pooled.reshape(_B, _M, 256), (0, 2, 1))
    return (cen_xyz, feat)
